# baseline (speedup 1.0000x reference)
import jax
import jax.numpy as jnp
from jax.experimental import pallas as pl
from jax.experimental.pallas import tpu as pltpu


def _fused_mlp(x_ref, w1_ref, b1_ref, w2_ref, b2_ref, w3_ref, b3_ref,
               wc_ref, f_ref, s_ref):
    h = jnp.dot(x_ref[...], w1_ref[...], preferred_element_type=jnp.float32)
    h = jnp.maximum(h + b1_ref[...], 0.0)
    h = jnp.dot(h, w2_ref[...], preferred_element_type=jnp.float32)
    h = jnp.maximum(h + b2_ref[...], 0.0)
    f = jnp.dot(h, w3_ref[...], preferred_element_type=jnp.float32)
    f = jnp.maximum(f + b3_ref[...], 0.0)
    f_ref[...] = f
    s_ref[...] = jnp.dot(f, wc_ref[...], preferred_element_type=jnp.float32)


def kernel(x, W1, b1, W2, b2, W3, b3, Wc):
    N, D = x.shape
    H1 = W1.shape[0]
    H2 = W2.shape[0]
    H3 = W3.shape[0]
    C = Wc.shape[0]

    BN = 1024

    W1t = W1.T
    W2t = W2.T
    W3t = W3.T
    Wct = Wc.T
    b1r = b1.reshape(1, H1)
    b2r = b2.reshape(1, H2)
    b3r = b3.reshape(1, H3)

    features, scores = pl.pallas_call(
        _fused_mlp,
        grid=(N // BN,),
        compiler_params=pltpu.CompilerParams(
            dimension_semantics=("parallel",),
        ),
        in_specs=[
            pl.BlockSpec((BN, D), lambda i: (i, 0)),
            pl.BlockSpec((D, H1), lambda i: (0, 0)),
            pl.BlockSpec((1, H1), lambda i: (0, 0)),
            pl.BlockSpec((H1, H2), lambda i: (0, 0)),
            pl.BlockSpec((1, H2), lambda i: (0, 0)),
            pl.BlockSpec((H2, H3), lambda i: (0, 0)),
            pl.BlockSpec((1, H3), lambda i: (0, 0)),
            pl.BlockSpec((H3, C), lambda i: (0, 0)),
        ],
        out_specs=[
            pl.BlockSpec((BN, H3), lambda i: (i, 0)),
            pl.BlockSpec((BN, C), lambda i: (i, 0)),
        ],
        out_shape=[
            jax.ShapeDtypeStruct((N, H3), jnp.float32),
            jax.ShapeDtypeStruct((N, C), jnp.float32),
        ],
    )(x, W1t, b1r, W2t, b2r, W3t, b3r, Wct)
    return (features, scores)


# speedup vs baseline: 1.4039x; 1.4039x over previous
import jax
import jax.numpy as jnp
from jax.experimental import pallas as pl
from jax.experimental.pallas import tpu as pltpu

_DN_T = (((1,), (1,)), ((), ()))


def _fused_mlp(x_ref, w1_ref, b1_ref, w2_ref, b2_ref, w3_ref, b3_ref,
               wc_ref, f_ref, s_ref):
    h = jax.lax.dot_general(x_ref[...], w1_ref[...], _DN_T,
                            preferred_element_type=jnp.float32)
    h = jnp.maximum(h + b1_ref[...], 0.0)
    h = jax.lax.dot_general(h, w2_ref[...], _DN_T,
                            preferred_element_type=jnp.float32)
    h = jnp.maximum(h + b2_ref[...], 0.0)
    f = jax.lax.dot_general(h, w3_ref[...], _DN_T,
                            preferred_element_type=jnp.float32)
    f = jnp.maximum(f + b3_ref[...], 0.0)
    f_ref[...] = f
    s_ref[...] = jax.lax.dot_general(f, wc_ref[...], _DN_T,
                                     preferred_element_type=jnp.float32)


def kernel(x, W1, b1, W2, b2, W3, b3, Wc):
    N, D = x.shape
    H1 = W1.shape[0]
    H2 = W2.shape[0]
    H3 = W3.shape[0]
    C = Wc.shape[0]

    BN = 2048

    features, scores = pl.pallas_call(
        _fused_mlp,
        grid=(N // BN,),
        compiler_params=pltpu.CompilerParams(
            dimension_semantics=("parallel",),
        ),
        in_specs=[
            pl.BlockSpec((BN, D), lambda i: (i, 0)),
            pl.BlockSpec((H1, D), lambda i: (0, 0)),
            pl.BlockSpec((H1,), lambda i: (0,)),
            pl.BlockSpec((H2, H1), lambda i: (0, 0)),
            pl.BlockSpec((H2,), lambda i: (0,)),
            pl.BlockSpec((H3, H2), lambda i: (0, 0)),
            pl.BlockSpec((H3,), lambda i: (0,)),
            pl.BlockSpec((C, H3), lambda i: (0, 0)),
        ],
        out_specs=[
            pl.BlockSpec((BN, H3), lambda i: (i, 0)),
            pl.BlockSpec((BN, C), lambda i: (i, 0)),
        ],
        out_shape=[
            jax.ShapeDtypeStruct((N, H3), jnp.float32),
            jax.ShapeDtypeStruct((N, C), jnp.float32),
        ],
    )(x, W1, b1, W2, b2, W3, b3, Wc)
    return (features, scores)


# speedup vs baseline: 1.5281x; 1.0885x over previous
import jax
import jax.numpy as jnp
from jax.experimental import pallas as pl
from jax.experimental.pallas import tpu as pltpu

_DN_T = (((1,), (1,)), ((), ()))


def _fused_mlp(x_ref, w1_ref, b1_ref, w2_ref, b2_ref, w3_ref, b3_ref,
               wc_ref, f_ref, s_ref):
    h = jax.lax.dot_general(x_ref[...], w1_ref[...], _DN_T,
                            preferred_element_type=jnp.float32)
    h = jnp.maximum(h + b1_ref[...], 0.0)
    h = jax.lax.dot_general(h, w2_ref[...], _DN_T,
                            preferred_element_type=jnp.float32)
    h = jnp.maximum(h + b2_ref[...], 0.0)
    f = jax.lax.dot_general(h, w3_ref[...], _DN_T,
                            preferred_element_type=jnp.float32)
    f = jnp.maximum(f + b3_ref[...], 0.0)
    f_ref[...] = f
    s_ref[...] = jax.lax.dot_general(f, wc_ref[...], _DN_T,
                                     preferred_element_type=jnp.float32)


def kernel(x, W1, b1, W2, b2, W3, b3, Wc):
    N, D = x.shape
    H1 = W1.shape[0]
    H2 = W2.shape[0]
    H3 = W3.shape[0]
    C = Wc.shape[0]

    BN = 4096

    features, scores = pl.pallas_call(
        _fused_mlp,
        grid=(N // BN,),
        compiler_params=pltpu.CompilerParams(
            dimension_semantics=("parallel",),
        ),
        in_specs=[
            pl.BlockSpec((BN, D), lambda i: (i, 0)),
            pl.BlockSpec((H1, D), lambda i: (0, 0)),
            pl.BlockSpec((H1,), lambda i: (0,)),
            pl.BlockSpec((H2, H1), lambda i: (0, 0)),
            pl.BlockSpec((H2,), lambda i: (0,)),
            pl.BlockSpec((H3, H2), lambda i: (0, 0)),
            pl.BlockSpec((H3,), lambda i: (0,)),
            pl.BlockSpec((C, H3), lambda i: (0, 0)),
        ],
        out_specs=[
            pl.BlockSpec((BN, H3), lambda i: (i, 0)),
            pl.BlockSpec((BN, C), lambda i: (i, 0)),
        ],
        out_shape=[
            jax.ShapeDtypeStruct((N, H3), jnp.float32),
            jax.ShapeDtypeStruct((N, C), jnp.float32),
        ],
    )(x, W1, b1, W2, b2, W3, b3, Wc)
    return (features, scores)
